# baseline (speedup 1.0000x reference)
import functools
import math

import jax
import jax.numpy as jnp
from jax import lax
from jax.experimental import pallas as pl
from jax.experimental.pallas import tpu as pltpu
from jax.experimental.pallas import tpu_sc as plsc

N = 10000
HN = N // 2
E = 320000
F_IN = 128
H = 64
NG = 64
BN_EPS = 1e-5
_BN_SCALE = 1.0 / math.sqrt(1.0 + BN_EPS)

NC = 2
NS = 16
NW = NC * NS
EPW = E // NW
K = 100
NCHUNK = EPW // K
G = 5
NGROUP = NCHUNK // G
RPT = 624
TAIL_BASE = NS * RPT
TAIL = N - TAIL_BASE
DEG_W = 16

_sc_mesh = plsc.VectorSubcoreMesh(core_axis_name="c", subcore_axis_name="s")
_sc_params = pltpu.CompilerParams(use_tc_tiling_on_sc=False)


@functools.partial(
    pl.kernel,
    out_type=(jax.ShapeDtypeStruct((N, H), jnp.float32),
              jax.ShapeDtypeStruct((N, H), jnp.float32)),
    mesh=_sc_mesh,
    scratch_types=[
        pltpu.VMEM((NCHUNK, K), jnp.int32),
        pltpu.VMEM((K, DEG_W), jnp.float32),
        pltpu.VMEM_SHARED((N, DEG_W), jnp.float32),
        pltpu.SemaphoreType.DMA,
        pltpu.SemaphoreType.DMA,
    ],
    compiler_params=_sc_params,
)
def _sc_degree(dst_hbm, zeros_hbm, out0_hbm, out1_hbm, dst_v, ones_v, acc,
               ssem, csem):
    cid = lax.axis_index("c")
    sid = lax.axis_index("s")
    wid = cid * NS + sid

    def _fill(i, carry):
        ones_v[i] = jnp.ones((DEG_W,), jnp.float32)
        return carry

    lax.fori_loop(0, K, _fill, 0)
    pltpu.async_copy(dst_hbm.at[wid], dst_v, csem)
    pltpu.async_copy(zeros_hbm.at[pl.ds(sid * RPT, RPT)],
                     acc.at[pl.ds(sid * RPT, RPT)], csem)
    pltpu.make_async_copy(dst_hbm.at[wid], dst_v, csem).wait()
    pltpu.make_async_copy(zeros_hbm.at[pl.ds(sid * RPT, RPT)],
                          acc.at[pl.ds(sid * RPT, RPT)], csem).wait()

    @pl.when(sid == NS - 1)
    def _zero_tail():
        pltpu.sync_copy(zeros_hbm.at[pl.ds(TAIL_BASE, TAIL)],
                        acc.at[pl.ds(TAIL_BASE, TAIL)])

    plsc.subcore_barrier()

    depth = 16

    def _body(j, carry):
        pltpu.async_copy(ones_v, acc.at[dst_v.at[j]], ssem, add=True)

        @pl.when(j >= depth)
        def _drain_one():
            pltpu.make_async_copy(ones_v, acc.at[dst_v.at[0]], ssem).wait()

        return carry

    lax.fori_loop(0, NCHUNK, _body, 0)

    def _drain(j, carry):
        pltpu.make_async_copy(ones_v, acc.at[dst_v.at[0]], ssem).wait()
        return carry

    lax.fori_loop(0, depth, _drain, 0)
    plsc.subcore_barrier()

    @pl.when(cid == 0)
    def _write0():
        pltpu.sync_copy(acc.at[pl.ds(sid * RPT, RPT)],
                        out0_hbm.at[pl.ds(sid * RPT, RPT), pl.ds(0, DEG_W)])

        @pl.when(sid == NS - 1)
        def _tail0():
            pltpu.sync_copy(acc.at[pl.ds(TAIL_BASE, TAIL)],
                            out0_hbm.at[pl.ds(TAIL_BASE, TAIL),
                                        pl.ds(0, DEG_W)])

    @pl.when(cid == 1)
    def _write1():
        pltpu.sync_copy(acc.at[pl.ds(sid * RPT, RPT)],
                        out1_hbm.at[pl.ds(sid * RPT, RPT), pl.ds(0, DEG_W)])

        @pl.when(sid == NS - 1)
        def _tail1():
            pltpu.sync_copy(acc.at[pl.ds(TAIL_BASE, TAIL)],
                            out1_hbm.at[pl.ds(TAIL_BASE, TAIL),
                                        pl.ds(0, DEG_W)])


@functools.partial(
    pl.kernel,
    out_type=(jax.ShapeDtypeStruct((N, H), jnp.float32),
              jax.ShapeDtypeStruct((N, H), jnp.float32)),
    mesh=_sc_mesh,
    scratch_types=[
        pltpu.VMEM((NCHUNK, K), jnp.int32),
        pltpu.VMEM((NCHUNK, K), jnp.int32),
        pltpu.VMEM((2, G, K, H), jnp.float32),
        pltpu.VMEM_SHARED((N, H), jnp.float32),
        pltpu.SemaphoreType.DMA,
        pltpu.SemaphoreType.DMA,
        pltpu.SemaphoreType.DMA,
    ],
    compiler_params=_sc_params,
)
def _sc_aggregate(hs_hbm, src_hbm, dst_hbm, zeros_hbm, out0_hbm, out1_hbm,
                  src_v, dst_v, rows_v, acc, gsem, ssem, csem):
    cid = lax.axis_index("c")
    sid = lax.axis_index("s")
    wid = cid * NS + sid

    pltpu.async_copy(src_hbm.at[wid], src_v, csem)
    pltpu.async_copy(dst_hbm.at[wid], dst_v, csem)
    pltpu.async_copy(zeros_hbm.at[pl.ds(sid * RPT, RPT)],
                     acc.at[pl.ds(sid * RPT, RPT)], csem)
    pltpu.make_async_copy(src_hbm.at[wid], src_v, csem).wait()
    pltpu.make_async_copy(dst_hbm.at[wid], dst_v, csem).wait()
    pltpu.make_async_copy(zeros_hbm.at[pl.ds(sid * RPT, RPT)],
                          acc.at[pl.ds(sid * RPT, RPT)], csem).wait()

    @pl.when(sid == NS - 1)
    def _zero_tail():
        pltpu.sync_copy(zeros_hbm.at[pl.ds(TAIL_BASE, TAIL)],
                        acc.at[pl.ds(TAIL_BASE, TAIL)])

    plsc.subcore_barrier()

    for b in range(G):
        pltpu.async_copy(hs_hbm.at[src_v.at[b]], rows_v.at[0, b], gsem)

    def _group(g, carry):
        bank = lax.rem(g, 2)

        for b in range(G):
            pltpu.make_async_copy(hs_hbm.at[src_v.at[g * G + b]],
                                  rows_v.at[bank, b], gsem).wait()
        for b in range(G):
            pltpu.async_copy(rows_v.at[bank, b],
                             acc.at[dst_v.at[g * G + b]], ssem, add=True)

        @pl.when(g > 0)
        def _drain_prev():
            for b in range(G):
                pltpu.make_async_copy(rows_v.at[1 - bank, b],
                                      acc.at[dst_v.at[0]], ssem).wait()

        @pl.when(g + 1 < NGROUP)
        def _prefetch():
            for b in range(G):
                pltpu.async_copy(hs_hbm.at[src_v.at[(g + 1) * G + b]],
                                 rows_v.at[1 - bank, b], gsem)

        return carry

    lax.fori_loop(0, NGROUP, _group, 0)
    for b in range(G):
        pltpu.make_async_copy(rows_v.at[0, 0],
                              acc.at[dst_v.at[0]], ssem).wait()
    plsc.subcore_barrier()

    @pl.when(cid == 0)
    def _write0():
        pltpu.sync_copy(acc.at[pl.ds(sid * RPT, RPT)],
                        out0_hbm.at[pl.ds(sid * RPT, RPT)])

        @pl.when(sid == NS - 1)
        def _tail0():
            pltpu.sync_copy(acc.at[pl.ds(TAIL_BASE, TAIL)],
                            out0_hbm.at[pl.ds(TAIL_BASE, TAIL)])

    @pl.when(cid == 1)
    def _write1():
        pltpu.sync_copy(acc.at[pl.ds(sid * RPT, RPT)],
                        out1_hbm.at[pl.ds(sid * RPT, RPT)])

        @pl.when(sid == NS - 1)
        def _tail1():
            pltpu.sync_copy(acc.at[pl.ds(TAIL_BASE, TAIL)],
                            out1_hbm.at[pl.ds(TAIL_BASE, TAIL)])


def _remap_body(e_ref, s_ref, d_ref):
    vs = e_ref[0:E // 128, :]
    vd = e_ref[E // 128:2 * E // 128, :]
    s_ref[...] = jnp.where(vs < HN, 2 * vs, 2 * (vs - HN) + 1)
    d_ref[...] = jnp.where(vd < HN, 2 * vd, 2 * (vd - HN) + 1)


def _mm1_body(x_ref, w_ref, o_ref):
    top = jnp.dot(x_ref[0:HN, :], w_ref[...],
                  preferred_element_type=jnp.float32)
    bot = jnp.dot(x_ref[HN:N, :], w_ref[...],
                  preferred_element_type=jnp.float32)
    o_ref[...] = jnp.concatenate([top, bot], axis=1)


def _scale_body(h_ref, d0_ref, d1_ref, rep_ref, hs_ref, dinv_ref):
    lane = lax.broadcasted_iota(jnp.int32, (HN, 2 * H), 1)
    d16 = jnp.where(lane % H < DEG_W, d0_ref[...] + d1_ref[...], 0.0)
    deg = jnp.dot(d16, rep_ref[...], preferred_element_type=jnp.float32)
    dinv = lax.rsqrt(deg + 1.0)
    dinv_ref[...] = dinv
    hs_ref[...] = h_ref[...] * dinv


def _mid_body(a0_ref, a1_ref, hs_ref, dinv_ref, b_ref, g_ref, be_ref,
              w_ref, o_ref):
    dinv = dinv_ref[...]
    t = (a0_ref[...] + a1_ref[...] + hs_ref[...]) * dinv + b_ref[...]
    t = t * (g_ref[...] * _BN_SCALE) + be_ref[...]
    t = jnp.maximum(t, 0.0)
    h = jnp.dot(t, w_ref[...], preferred_element_type=jnp.float32)
    o_ref[...] = h * dinv


def _final_body(a0_ref, a1_ref, hs_ref, dinv_ref, b_ref, g_ref, be_ref,
                bev_ref, bod_ref, lw1_ref, lb1_ref, lw2_ref, lb2_ref, o_ref):
    t = (a0_ref[...] + a1_ref[...] + hs_ref[...]) * dinv_ref[...] + b_ref[...]
    t = t * (g_ref[...] * _BN_SCALE) + be_ref[...]
    t = jnp.maximum(t, 0.0)
    gids = lax.broadcasted_iota(jnp.int32, (NG, HN), 0)
    mask_e = (bev_ref[...] == gids).astype(jnp.float32)
    mask_o = (bod_ref[...] == gids).astype(jnp.float32)
    pe = jnp.dot(mask_e, t, preferred_element_type=jnp.float32)
    po = jnp.dot(mask_o, t, preferred_element_type=jnp.float32)
    pooled = pe[:, 0:H] + po[:, H:2 * H]
    z = jnp.dot(pooled, lw1_ref[...], preferred_element_type=jnp.float32)
    z = jnp.maximum(z + lb1_ref[...], 0.0)
    o_ref[...] = (jnp.dot(z, lw2_ref[...], preferred_element_type=jnp.float32)
                  + lb2_ref[...])


def _tc(body, out_shape, *args):
    return pl.pallas_call(body, out_shape=out_shape)(*args)


def _pairc(v):
    return jnp.concatenate([v, v]).reshape(1, 2 * H)


def _blockdiag(w):
    z = jnp.zeros((H, H), jnp.float32)
    return jnp.concatenate([
        jnp.concatenate([w, z], axis=1),
        jnp.concatenate([z, w], axis=1)], axis=0)


def kernel(x, edge_index, batch, W1, b1, W2, b2, W3, b3,
           g1, be1, g2, be2, g3, be3, lw1, lb1, lw2, lb2):
    f32 = jnp.float32
    sds = jax.ShapeDtypeStruct

    eidx = edge_index.reshape(2 * E // 128, 128)
    src, dst = _tc(_remap_body, (sds((E // 128, 128), jnp.int32),
                                 sds((E // 128, 128), jnp.int32)), eidx)
    src = src.reshape(NW, NCHUNK, K)
    dst = dst.reshape(NW, NCHUNK, K)

    zeros_h = jnp.zeros((N, H), f32)
    zeros_d = jnp.zeros((N, DEG_W), f32)
    bev = batch[:HN].reshape(1, HN)
    bod = batch[HN:].reshape(1, HN)
    b1p, b2p, b3p = _pairc(b1), _pairc(b2), _pairc(b3)
    g1p, g2p, g3p = _pairc(g1), _pairc(g2), _pairc(g3)
    be1p, be2p, be3p = _pairc(be1), _pairc(be2), _pairc(be3)
    W2d, W3d = _blockdiag(W2), _blockdiag(W3)
    lb1r = lb1.reshape(1, H // 2)
    lb2r = lb2.reshape(1, 2)

    li = jnp.arange(2 * H)
    rep = ((li[:, None] // H == li[None, :] // H)
           & (li[:, None] % H < DEG_W)).astype(f32) / DEG_W

    h1 = _tc(_mm1_body, sds((HN, 2 * H), f32), x, W1)
    dg0, dg1 = _sc_degree(dst, zeros_d)
    hs1, dinv = _tc(_scale_body, (sds((HN, 2 * H), f32),
                                  sds((HN, 2 * H), f32)),
                    h1, dg0.reshape(HN, 2 * H), dg1.reshape(HN, 2 * H), rep)

    a10, a11 = _sc_aggregate(hs1.reshape(N, H), src, dst, zeros_h)
    hs2 = _tc(_mid_body, sds((HN, 2 * H), f32),
              a10.reshape(HN, 2 * H), a11.reshape(HN, 2 * H),
              hs1, dinv, b1p, g1p, be1p, W2d)

    a20, a21 = _sc_aggregate(hs2.reshape(N, H), src, dst, zeros_h)
    hs3 = _tc(_mid_body, sds((HN, 2 * H), f32),
              a20.reshape(HN, 2 * H), a21.reshape(HN, 2 * H),
              hs2, dinv, b2p, g2p, be2p, W3d)

    a30, a31 = _sc_aggregate(hs3.reshape(N, H), src, dst, zeros_h)
    out = _tc(_final_body, sds((NG, 2), f32),
              a30.reshape(HN, 2 * H), a31.reshape(HN, 2 * H),
              hs3, dinv, b3p, g3p, be3p, bev, bod,
              lw1, lb1r, lw2, lb2r)
    return out


# speedup vs baseline: 1.0048x; 1.0048x over previous
import functools
import math

import jax
import jax.numpy as jnp
from jax import lax
from jax.experimental import pallas as pl
from jax.experimental.pallas import tpu as pltpu
from jax.experimental.pallas import tpu_sc as plsc

N = 10000
HN = N // 2
E = 320000
F_IN = 128
H = 64
NG = 64
BN_EPS = 1e-5
_BN_SCALE = 1.0 / math.sqrt(1.0 + BN_EPS)

NC = 2
NS = 16
NW = NC * NS
EPW = E // NW
K = 80
NCHUNK = EPW // K
G = 5
NGROUP = NCHUNK // G
RPT = 624
TAIL_BASE = NS * RPT
TAIL = N - TAIL_BASE
DEG_W = 16

_sc_mesh = plsc.VectorSubcoreMesh(core_axis_name="c", subcore_axis_name="s")
_sc_params = pltpu.CompilerParams(use_tc_tiling_on_sc=False)


@functools.partial(
    pl.kernel,
    out_type=(jax.ShapeDtypeStruct((N, H), jnp.float32),
              jax.ShapeDtypeStruct((N, H), jnp.float32)),
    mesh=_sc_mesh,
    scratch_types=[
        pltpu.VMEM((NCHUNK, K), jnp.int32),
        pltpu.VMEM((K, DEG_W), jnp.float32),
        pltpu.VMEM_SHARED((N, DEG_W), jnp.float32),
        pltpu.SemaphoreType.DMA,
        pltpu.SemaphoreType.DMA,
    ],
    compiler_params=_sc_params,
)
def _sc_degree(dst_hbm, zeros_hbm, out0_hbm, out1_hbm, dst_v, ones_v, acc,
               ssem, csem):
    cid = lax.axis_index("c")
    sid = lax.axis_index("s")
    wid = cid * NS + sid

    def _fill(i, carry):
        ones_v[i] = jnp.ones((DEG_W,), jnp.float32)
        return carry

    lax.fori_loop(0, K, _fill, 0)
    pltpu.async_copy(dst_hbm.at[wid], dst_v, csem)
    pltpu.async_copy(zeros_hbm.at[pl.ds(sid * RPT, RPT)],
                     acc.at[pl.ds(sid * RPT, RPT)], csem)
    pltpu.make_async_copy(dst_hbm.at[wid], dst_v, csem).wait()
    pltpu.make_async_copy(zeros_hbm.at[pl.ds(sid * RPT, RPT)],
                          acc.at[pl.ds(sid * RPT, RPT)], csem).wait()

    @pl.when(sid == NS - 1)
    def _zero_tail():
        pltpu.sync_copy(zeros_hbm.at[pl.ds(TAIL_BASE, TAIL)],
                        acc.at[pl.ds(TAIL_BASE, TAIL)])

    plsc.subcore_barrier()

    depth = 16

    def _body(j, carry):
        pltpu.async_copy(ones_v, acc.at[dst_v.at[j]], ssem, add=True)

        @pl.when(j >= depth)
        def _drain_one():
            pltpu.make_async_copy(ones_v, acc.at[dst_v.at[0]], ssem).wait()

        return carry

    lax.fori_loop(0, NCHUNK, _body, 0)

    def _drain(j, carry):
        pltpu.make_async_copy(ones_v, acc.at[dst_v.at[0]], ssem).wait()
        return carry

    lax.fori_loop(0, depth, _drain, 0)
    plsc.subcore_barrier()

    @pl.when(cid == 0)
    def _write0():
        pltpu.sync_copy(acc.at[pl.ds(sid * RPT, RPT)],
                        out0_hbm.at[pl.ds(sid * RPT, RPT), pl.ds(0, DEG_W)])

        @pl.when(sid == NS - 1)
        def _tail0():
            pltpu.sync_copy(acc.at[pl.ds(TAIL_BASE, TAIL)],
                            out0_hbm.at[pl.ds(TAIL_BASE, TAIL),
                                        pl.ds(0, DEG_W)])

    @pl.when(cid == 1)
    def _write1():
        pltpu.sync_copy(acc.at[pl.ds(sid * RPT, RPT)],
                        out1_hbm.at[pl.ds(sid * RPT, RPT), pl.ds(0, DEG_W)])

        @pl.when(sid == NS - 1)
        def _tail1():
            pltpu.sync_copy(acc.at[pl.ds(TAIL_BASE, TAIL)],
                            out1_hbm.at[pl.ds(TAIL_BASE, TAIL),
                                        pl.ds(0, DEG_W)])


@functools.partial(
    pl.kernel,
    out_type=(jax.ShapeDtypeStruct((N, H), jnp.float32),
              jax.ShapeDtypeStruct((N, H), jnp.float32)),
    mesh=_sc_mesh,
    scratch_types=[
        pltpu.VMEM((NCHUNK, K), jnp.int32),
        pltpu.VMEM((NCHUNK, K), jnp.int32),
        pltpu.VMEM((2, G, K, H), jnp.float32),
        pltpu.VMEM_SHARED((N, H), jnp.float32),
        pltpu.SemaphoreType.DMA,
        pltpu.SemaphoreType.DMA,
        pltpu.SemaphoreType.DMA,
    ],
    compiler_params=_sc_params,
)
def _sc_aggregate(hs_hbm, src_hbm, dst_hbm, zeros_hbm, out0_hbm, out1_hbm,
                  src_v, dst_v, rows_v, acc, gsem, ssem, csem):
    cid = lax.axis_index("c")
    sid = lax.axis_index("s")
    wid = cid * NS + sid

    pltpu.async_copy(src_hbm.at[wid], src_v, csem)
    pltpu.async_copy(dst_hbm.at[wid], dst_v, csem)
    pltpu.async_copy(zeros_hbm.at[pl.ds(sid * RPT, RPT)],
                     acc.at[pl.ds(sid * RPT, RPT)], csem)
    pltpu.make_async_copy(src_hbm.at[wid], src_v, csem).wait()
    pltpu.make_async_copy(dst_hbm.at[wid], dst_v, csem).wait()
    pltpu.make_async_copy(zeros_hbm.at[pl.ds(sid * RPT, RPT)],
                          acc.at[pl.ds(sid * RPT, RPT)], csem).wait()

    @pl.when(sid == NS - 1)
    def _zero_tail():
        pltpu.sync_copy(zeros_hbm.at[pl.ds(TAIL_BASE, TAIL)],
                        acc.at[pl.ds(TAIL_BASE, TAIL)])

    plsc.subcore_barrier()

    for b in range(G):
        pltpu.async_copy(hs_hbm.at[src_v.at[b]], rows_v.at[0, b], gsem)

    def _group(g, carry):
        bank = lax.rem(g, 2)

        for b in range(G):
            pltpu.make_async_copy(hs_hbm.at[src_v.at[g * G + b]],
                                  rows_v.at[bank, b], gsem).wait()
        for b in range(G):
            pltpu.async_copy(rows_v.at[bank, b],
                             acc.at[dst_v.at[g * G + b]], ssem, add=True)

        @pl.when(g > 0)
        def _drain_prev():
            for b in range(G):
                pltpu.make_async_copy(rows_v.at[1 - bank, b],
                                      acc.at[dst_v.at[0]], ssem).wait()

        @pl.when(g + 1 < NGROUP)
        def _prefetch():
            for b in range(G):
                pltpu.async_copy(hs_hbm.at[src_v.at[(g + 1) * G + b]],
                                 rows_v.at[1 - bank, b], gsem)

        return carry

    lax.fori_loop(0, NGROUP, _group, 0)
    for b in range(G):
        pltpu.make_async_copy(rows_v.at[0, 0],
                              acc.at[dst_v.at[0]], ssem).wait()
    plsc.subcore_barrier()

    @pl.when(cid == 0)
    def _write0():
        pltpu.sync_copy(acc.at[pl.ds(sid * RPT, RPT)],
                        out0_hbm.at[pl.ds(sid * RPT, RPT)])

        @pl.when(sid == NS - 1)
        def _tail0():
            pltpu.sync_copy(acc.at[pl.ds(TAIL_BASE, TAIL)],
                            out0_hbm.at[pl.ds(TAIL_BASE, TAIL)])

    @pl.when(cid == 1)
    def _write1():
        pltpu.sync_copy(acc.at[pl.ds(sid * RPT, RPT)],
                        out1_hbm.at[pl.ds(sid * RPT, RPT)])

        @pl.when(sid == NS - 1)
        def _tail1():
            pltpu.sync_copy(acc.at[pl.ds(TAIL_BASE, TAIL)],
                            out1_hbm.at[pl.ds(TAIL_BASE, TAIL)])


def _remap_body(e_ref, s_ref, d_ref):
    vs = e_ref[0:E // 128, :]
    vd = e_ref[E // 128:2 * E // 128, :]
    s_ref[...] = jnp.where(vs < HN, 2 * vs, 2 * (vs - HN) + 1)
    d_ref[...] = jnp.where(vd < HN, 2 * vd, 2 * (vd - HN) + 1)


def _mm1_body(x_ref, w_ref, o_ref):
    top = jnp.dot(x_ref[0:HN, :], w_ref[...],
                  preferred_element_type=jnp.float32)
    bot = jnp.dot(x_ref[HN:N, :], w_ref[...],
                  preferred_element_type=jnp.float32)
    o_ref[...] = jnp.concatenate([top, bot], axis=1)


def _scale_body(h_ref, d0_ref, d1_ref, rep_ref, hs_ref, dinv_ref):
    lane = lax.broadcasted_iota(jnp.int32, (HN, 2 * H), 1)
    d16 = jnp.where(lane % H < DEG_W, d0_ref[...] + d1_ref[...], 0.0)
    deg = jnp.dot(d16, rep_ref[...], preferred_element_type=jnp.float32)
    dinv = lax.rsqrt(deg + 1.0)
    dinv_ref[...] = dinv
    hs_ref[...] = h_ref[...] * dinv


def _mid_body(a0_ref, a1_ref, hs_ref, dinv_ref, b_ref, g_ref, be_ref,
              w_ref, o_ref):
    dinv = dinv_ref[...]
    t = (a0_ref[...] + a1_ref[...] + hs_ref[...]) * dinv + b_ref[...]
    t = t * (g_ref[...] * _BN_SCALE) + be_ref[...]
    t = jnp.maximum(t, 0.0)
    h = jnp.dot(t, w_ref[...], preferred_element_type=jnp.float32)
    o_ref[...] = h * dinv


def _final_body(a0_ref, a1_ref, hs_ref, dinv_ref, b_ref, g_ref, be_ref,
                bev_ref, bod_ref, lw1_ref, lb1_ref, lw2_ref, lb2_ref, o_ref):
    t = (a0_ref[...] + a1_ref[...] + hs_ref[...]) * dinv_ref[...] + b_ref[...]
    t = t * (g_ref[...] * _BN_SCALE) + be_ref[...]
    t = jnp.maximum(t, 0.0)
    gids = lax.broadcasted_iota(jnp.int32, (NG, HN), 0)
    mask_e = (bev_ref[...] == gids).astype(jnp.float32)
    mask_o = (bod_ref[...] == gids).astype(jnp.float32)
    pe = jnp.dot(mask_e, t, preferred_element_type=jnp.float32)
    po = jnp.dot(mask_o, t, preferred_element_type=jnp.float32)
    pooled = pe[:, 0:H] + po[:, H:2 * H]
    z = jnp.dot(pooled, lw1_ref[...], preferred_element_type=jnp.float32)
    z = jnp.maximum(z + lb1_ref[...], 0.0)
    o_ref[...] = (jnp.dot(z, lw2_ref[...], preferred_element_type=jnp.float32)
                  + lb2_ref[...])


def _tc(body, out_shape, *args):
    return pl.pallas_call(body, out_shape=out_shape)(*args)


def _pairc(v):
    return jnp.concatenate([v, v]).reshape(1, 2 * H)


def _blockdiag(w):
    z = jnp.zeros((H, H), jnp.float32)
    return jnp.concatenate([
        jnp.concatenate([w, z], axis=1),
        jnp.concatenate([z, w], axis=1)], axis=0)


def kernel(x, edge_index, batch, W1, b1, W2, b2, W3, b3,
           g1, be1, g2, be2, g3, be3, lw1, lb1, lw2, lb2):
    f32 = jnp.float32
    sds = jax.ShapeDtypeStruct

    eidx = edge_index.reshape(2 * E // 128, 128)
    src, dst = _tc(_remap_body, (sds((E // 128, 128), jnp.int32),
                                 sds((E // 128, 128), jnp.int32)), eidx)
    src = src.reshape(NW, NCHUNK, K)
    dst = dst.reshape(NW, NCHUNK, K)

    zeros_h = jnp.zeros((N, H), f32)
    zeros_d = jnp.zeros((N, DEG_W), f32)
    bev = batch[:HN].reshape(1, HN)
    bod = batch[HN:].reshape(1, HN)
    b1p, b2p, b3p = _pairc(b1), _pairc(b2), _pairc(b3)
    g1p, g2p, g3p = _pairc(g1), _pairc(g2), _pairc(g3)
    be1p, be2p, be3p = _pairc(be1), _pairc(be2), _pairc(be3)
    W2d, W3d = _blockdiag(W2), _blockdiag(W3)
    lb1r = lb1.reshape(1, H // 2)
    lb2r = lb2.reshape(1, 2)

    li = jnp.arange(2 * H)
    rep = ((li[:, None] // H == li[None, :] // H)
           & (li[:, None] % H < DEG_W)).astype(f32) / DEG_W

    h1 = _tc(_mm1_body, sds((HN, 2 * H), f32), x, W1)
    dg0, dg1 = _sc_degree(dst, zeros_d)
    hs1, dinv = _tc(_scale_body, (sds((HN, 2 * H), f32),
                                  sds((HN, 2 * H), f32)),
                    h1, dg0.reshape(HN, 2 * H), dg1.reshape(HN, 2 * H), rep)

    a10, a11 = _sc_aggregate(hs1.reshape(N, H), src, dst, zeros_h)
    hs2 = _tc(_mid_body, sds((HN, 2 * H), f32),
              a10.reshape(HN, 2 * H), a11.reshape(HN, 2 * H),
              hs1, dinv, b1p, g1p, be1p, W2d)

    a20, a21 = _sc_aggregate(hs2.reshape(N, H), src, dst, zeros_h)
    hs3 = _tc(_mid_body, sds((HN, 2 * H), f32),
              a20.reshape(HN, 2 * H), a21.reshape(HN, 2 * H),
              hs2, dinv, b2p, g2p, be2p, W3d)

    a30, a31 = _sc_aggregate(hs3.reshape(N, H), src, dst, zeros_h)
    out = _tc(_final_body, sds((NG, 2), f32),
              a30.reshape(HN, 2 * H), a31.reshape(HN, 2 * H),
              hs3, dinv, b3p, g3p, be3p, bev, bod,
              lw1, lb1r, lw2, lb2r)
    return out
